# baseline (speedup 1.0000x reference)
import functools

import jax
import jax.numpy as jnp
from jax.experimental import pallas as pl
from jax.experimental.pallas import tpu as pltpu

B, S, D, E = 4, 4096, 2048, 64
TM = 512
N = (B * S) // TM
NBUF = 4
LOOKAHEAD = NBUF - 1


def _copy(x_hbm, xbuf, sem, tile):
    slot = tile % NBUF
    return pltpu.make_async_copy(
        x_hbm.at[pl.ds(tile * TM, TM), :], xbuf.at[slot], sem.at[slot])


def _router_kernel(x_hbm, w_ref, sm_ref, idx_ref, xbuf, sem):
    i = pl.program_id(0)

    @pl.when(i == 0)
    def _prime():
        for t in range(LOOKAHEAD):
            _copy(x_hbm, xbuf, sem, t).start()

    @pl.when(i + LOOKAHEAD < N)
    def _lookahead():
        _copy(x_hbm, xbuf, sem, i + LOOKAHEAD).start()

    _copy(x_hbm, xbuf, sem, i).wait()

    x = xbuf[i % NBUF]
    w = w_ref[...]
    logits = jax.lax.dot_general(
        x, w, dimension_numbers=(((1,), (1,)), ((), ())),
        preferred_element_type=jnp.float32)
    m = jnp.max(logits, axis=-1, keepdims=True)
    e = jnp.exp(logits - m)
    sm = e / jnp.sum(e, axis=-1, keepdims=True)
    sm_ref[...] = sm
    idx_ref[...] = jnp.argmax(sm, axis=-1, keepdims=True).astype(jnp.int32)


@functools.partial(jax.jit, static_argnames=())
def kernel(inputs, W):
    T = B * S
    x = inputs.reshape(T, D)
    sm, idx = pl.pallas_call(
        _router_kernel,
        grid=(N,),
        in_specs=[
            pl.BlockSpec(memory_space=pltpu.MemorySpace.HBM),
            pl.BlockSpec((E, D), lambda i: (0, 0)),
        ],
        out_specs=[
            pl.BlockSpec((TM, E), lambda i: (i, 0)),
            pl.BlockSpec((TM, 1), lambda i: (i, 0)),
        ],
        out_shape=[
            jax.ShapeDtypeStruct((T, E), jnp.float32),
            jax.ShapeDtypeStruct((T, 1), jnp.int32),
        ],
        scratch_shapes=[
            pltpu.VMEM((NBUF, TM, D), jnp.float32),
            pltpu.SemaphoreType.DMA((NBUF,)),
        ],
        compiler_params=pltpu.CompilerParams(
            dimension_semantics=("arbitrary",),
        ),
    )(x, W)
    return idx.reshape(B, S), sm.reshape(B, S, E)
